# Initial kernel scaffold; baseline (speedup 1.0000x reference)
import jax
import jax.numpy as jnp
from jax.experimental import pallas as pl


def kernel(x, edge1, edge2, pos1, pos2, emb1_w, emb2_w, gn_e1_w, gn_e1_b, gn_e1_ms, gn_e2_w, gn_e2_b, gn_e2_ms, W1, b1, gn1_w, gn1_b, gn1_ms, W2, b2, gn2_w, gn2_b, gn2_ms, Wp, bp):
    raise NotImplementedError("write your pallas kernel here")


# speedup vs baseline: 1.8323x; 1.8323x over previous
import functools

import jax
import jax.numpy as jnp
from jax import lax
from jax.experimental import pallas as pl
from jax.experimental.pallas import tpu as pltpu

N = 100000
E = 1600000
P = 100000
Q = 10000
NB = 100
BN = 1000
EPS = 1e-5


def _prep_body(x_ref, dega_ref, degb_ref, cntx_ref, emb_ref, gw_ref, gb_ref,
               gms_ref, w1_ref, gab_ref, dis_ref):
    cnt = cntx_ref[0:1, :] + cntx_ref[1:2, :]
    p = cnt / N
    emb = emb_ref[...]
    mean = p @ emb
    ctr = emb - mean * gms_ref[...]
    var = p @ (ctr * ctr)
    tbl = ctr * (gw_ref[...] * lax.rsqrt(var + EPS)) + gb_ref[...]
    t1 = tbl @ w1_ref[...]

    deg = dega_ref[0] + degb_ref[0] + 1.0
    dis = lax.rsqrt(deg)
    dis_ref[0] = dis

    xb = x_ref[0]
    iota = lax.broadcasted_iota(jnp.int32, (BN, 128), 1)
    oh = (xb == iota).astype(jnp.float32)
    g32 = (oh @ t1) * dis
    gab_ref[0, 0] = g32[:, :16]
    gab_ref[1, 0] = g32[:, 16:]


def _tc_prep(x3, dega, degb, cntx, emb1p, gw, gb, gms, w1):
    return pl.pallas_call(
        _prep_body,
        grid=(NB,),
        in_specs=[
            pl.BlockSpec((1, BN, 1), lambda i: (i, 0, 0)),
            pl.BlockSpec((1, BN, 1), lambda i: (i, 0, 0)),
            pl.BlockSpec((1, BN, 1), lambda i: (i, 0, 0)),
            pl.BlockSpec((2, 128), lambda i: (0, 0)),
            pl.BlockSpec((128, 32), lambda i: (0, 0)),
            pl.BlockSpec((1, 32), lambda i: (0, 0)),
            pl.BlockSpec((1, 32), lambda i: (0, 0)),
            pl.BlockSpec((1, 32), lambda i: (0, 0)),
            pl.BlockSpec((32, 32), lambda i: (0, 0)),
        ],
        out_specs=[
            pl.BlockSpec((2, 1, BN, 16), lambda i: (0, i, 0, 0)),
            pl.BlockSpec((1, BN, 1), lambda i: (i, 0, 0)),
        ],
        out_shape=[
            jax.ShapeDtypeStruct((2, NB, BN, 16), jnp.float32),
            jax.ShapeDtypeStruct((NB, BN, 1), jnp.float32),
        ],
    )(x3, dega, degb, cntx, emb1p, gw, gb, gms, w1)


def _epi1_body(acc_ref, g_ref, dis_ref, b_ref, gw_ref, gb_ref, gms_ref,
               ma_ref, mb_ref, hn_ref, c_ref, ssum, ssq, msum):
    ph = pl.program_id(0)
    i = pl.program_id(1)
    nn = jnp.float32(N)

    pre = dis_ref[0] * (acc_ref[:, 0] + g_ref[:, 0]) + b_ref[...][:, None, :]

    @pl.when(jnp.logical_and(ph == 0, i == 0))
    def _init():
        ssum[...] = jnp.zeros_like(ssum)
        ssq[...] = jnp.zeros_like(ssq)
        msum[0, 0] = 0.0

    @pl.when(ph == 0)
    def _accum():
        ssum[...] += jnp.sum(pre, axis=1)
        ssq[...] += jnp.sum(pre * pre, axis=1)
        m = jnp.minimum(ma_ref[0] + mb_ref[0], 1.0)
        msum[0, 0] += jnp.sum(m)

    @pl.when(ph == 1)
    def _apply():
        mean = ssum[...] / nn
        mms = mean * gms_ref[...]
        var = ssq[...] / nn - 2.0 * mms * mean + mms * mms
        scale = gw_ref[...] * lax.rsqrt(var + EPS)
        hn = jnp.maximum((pre - mms[:, None, :]) * scale[:, None, :]
                         + gb_ref[...][:, None, :], 0.0)
        hn_ref[:, 0] = hn

    c_ref[...] = jnp.full((1, 1), msum[0, 0], jnp.float32)


def _tc_epi(acc, g, dis, b2d, gw2d, gb2d, gms2d, ma, mb):
    return pl.pallas_call(
        _epi1_body,
        grid=(2, NB),
        in_specs=[
            pl.BlockSpec((2, 1, BN, 16), lambda p, i: (0, i, 0, 0)),
            pl.BlockSpec((2, 1, BN, 16), lambda p, i: (0, i, 0, 0)),
            pl.BlockSpec((1, BN, 1), lambda p, i: (i, 0, 0)),
            pl.BlockSpec((2, 16), lambda p, i: (0, 0)),
            pl.BlockSpec((2, 16), lambda p, i: (0, 0)),
            pl.BlockSpec((2, 16), lambda p, i: (0, 0)),
            pl.BlockSpec((2, 16), lambda p, i: (0, 0)),
            pl.BlockSpec((1, BN, 1), lambda p, i: (i, 0, 0)),
            pl.BlockSpec((1, BN, 1), lambda p, i: (i, 0, 0)),
        ],
        out_specs=[
            pl.BlockSpec((2, 1, BN, 16), lambda p, i: (0, i, 0, 0)),
            pl.BlockSpec((1, 1), lambda p, i: (0, 0)),
        ],
        out_shape=[
            jax.ShapeDtypeStruct((2, NB, BN, 16), jnp.float32),
            jax.ShapeDtypeStruct((1, 1), jnp.float32),
        ],
        scratch_shapes=[
            pltpu.VMEM((2, 16), jnp.float32),
            pltpu.VMEM((2, 16), jnp.float32),
            pltpu.SMEM((1, 1), jnp.float32),
        ],
    )(acc, g, dis, b2d, gw2d, gb2d, gms2d, ma, mb)


def _mid_body(h2_ref, ma_ref, mb_ref, c_ref, d2a_ref, d2b_ref, emb2_ref,
              zw_ref, zb_ref, zms_ref, w2_ref, g2_ref, dis2_ref):
    pp = jnp.float32(P)
    c = c_ref[...]
    pz1 = c / pp
    pz0 = 1.0 - pz1
    e = emb2_ref[...]
    meanz = pz0 * e[0:1] + pz1 * e[1:2]
    ctr = e - meanz * zms_ref[...]
    varz = pz0 * (ctr[0:1] * ctr[0:1]) + pz1 * (ctr[1:2] * ctr[1:2])
    ztbl = ctr * (zw_ref[...] * lax.rsqrt(varz + EPS)) + zb_ref[...]
    zrow = ztbl @ w2_ref[32:, :]

    h2cat = jnp.concatenate([h2_ref[0, 0], h2_ref[1, 0]], axis=-1)
    m = jnp.minimum(ma_ref[0] + mb_ref[0], 1.0)
    u = h2cat @ w2_ref[:32, :] + zrow[0:1] + m * (zrow[1:2] - zrow[0:1])

    deg = d2a_ref[0] + d2b_ref[0] + 1.0
    dis = lax.rsqrt(deg)
    dis2_ref[0] = dis
    g2 = u * dis
    g2_ref[0, 0] = g2[:, :16]
    g2_ref[1, 0] = g2[:, 16:]


def _tc_mid(h2ab, ma, mb, c_arr, d2a, d2b, emb2, zw, zb, zms, w2):
    return pl.pallas_call(
        _mid_body,
        grid=(NB,),
        in_specs=[
            pl.BlockSpec((2, 1, BN, 16), lambda i: (0, i, 0, 0)),
            pl.BlockSpec((1, BN, 1), lambda i: (i, 0, 0)),
            pl.BlockSpec((1, BN, 1), lambda i: (i, 0, 0)),
            pl.BlockSpec((1, 1), lambda i: (0, 0)),
            pl.BlockSpec((1, BN, 1), lambda i: (i, 0, 0)),
            pl.BlockSpec((1, BN, 1), lambda i: (i, 0, 0)),
            pl.BlockSpec((2, 16), lambda i: (0, 0)),
            pl.BlockSpec((1, 16), lambda i: (0, 0)),
            pl.BlockSpec((1, 16), lambda i: (0, 0)),
            pl.BlockSpec((1, 16), lambda i: (0, 0)),
            pl.BlockSpec((48, 32), lambda i: (0, 0)),
        ],
        out_specs=[
            pl.BlockSpec((2, 1, BN, 16), lambda i: (0, i, 0, 0)),
            pl.BlockSpec((1, BN, 1), lambda i: (i, 0, 0)),
        ],
        out_shape=[
            jax.ShapeDtypeStruct((2, NB, BN, 16), jnp.float32),
            jax.ShapeDtypeStruct((NB, BN, 1), jnp.float32),
        ],
    )(h2ab, ma, mb, c_arr, d2a, d2b, emb2, zw, zb, zms, w2)


def _epi2_body(acc_ref, g_ref, dis_ref, b_ref, gw_ref, gb_ref, gms_ref,
               wp_ref, bp_ref, proj_ref, ssum, ssq):
    ph = pl.program_id(0)
    i = pl.program_id(1)
    nn = jnp.float32(P)

    pre = dis_ref[0] * (acc_ref[:, 0] + g_ref[:, 0]) + b_ref[...][:, None, :]

    @pl.when(jnp.logical_and(ph == 0, i == 0))
    def _init():
        ssum[...] = jnp.zeros_like(ssum)
        ssq[...] = jnp.zeros_like(ssq)

    @pl.when(ph == 0)
    def _accum():
        ssum[...] += jnp.sum(pre, axis=1)
        ssq[...] += jnp.sum(pre * pre, axis=1)

    @pl.when(ph == 1)
    def _apply():
        mean = ssum[...] / nn
        mms = mean * gms_ref[...]
        var = ssq[...] / nn - 2.0 * mms * mean + mms * mms
        scale = gw_ref[...] * lax.rsqrt(var + EPS)
        hf = jnp.maximum((pre - mms[:, None, :]) * scale[:, None, :]
                         + gb_ref[...][:, None, :], 0.0)
        hcat = jnp.concatenate([hf[0], hf[1]], axis=-1)
        proj_ref[0] = hcat @ wp_ref[...] + bp_ref[...]


def _tc_epi2(acc, g, dis, b2d, gw2d, gb2d, gms2d, wp, bp):
    return pl.pallas_call(
        _epi2_body,
        grid=(2, NB),
        in_specs=[
            pl.BlockSpec((2, 1, BN, 16), lambda p, i: (0, i, 0, 0)),
            pl.BlockSpec((2, 1, BN, 16), lambda p, i: (0, i, 0, 0)),
            pl.BlockSpec((1, BN, 1), lambda p, i: (i, 0, 0)),
            pl.BlockSpec((2, 16), lambda p, i: (0, 0)),
            pl.BlockSpec((2, 16), lambda p, i: (0, 0)),
            pl.BlockSpec((2, 16), lambda p, i: (0, 0)),
            pl.BlockSpec((2, 16), lambda p, i: (0, 0)),
            pl.BlockSpec((32, 1), lambda p, i: (0, 0)),
            pl.BlockSpec((1, 1), lambda p, i: (0, 0)),
        ],
        out_specs=[
            pl.BlockSpec((1, BN, 1), lambda p, i: (i, 0, 0)),
        ],
        out_shape=[
            jax.ShapeDtypeStruct((NB, BN, 1), jnp.float32),
        ],
        scratch_shapes=[
            pltpu.VMEM((2, 16), jnp.float32),
            pltpu.VMEM((2, 16), jnp.float32),
        ],
    )(acc, g, dis, b2d, gw2d, gb2d, gms2d, wp, bp)


def _r2d(v):
    return v.reshape(2, 16)


def kernel(x, edge1, edge2, pos1, pos2, emb1_w, emb2_w, gn_e1_w, gn_e1_b,
           gn_e1_ms, gn_e2_w, gn_e2_b, gn_e2_ms, W1, b1, gn1_w, gn1_b,
           gn1_ms, W2, b2, gn2_w, gn2_b, gn2_ms, Wp, bp):
    x = x.astype(jnp.int32)
    edge1 = edge1.astype(jnp.int32)
    edge2 = edge2.astype(jnp.int32)
    pos1 = pos1.astype(jnp.int32)
    pos2 = pos2.astype(jnp.int32)

    cntx = jnp.zeros((128,), jnp.float32).at[x].add(1.0)
    deg1 = jnp.zeros((N,), jnp.float32).at[edge1[1]].add(1.0)
    deg2 = jnp.zeros((P,), jnp.float32).at[edge2[1]].add(1.0)
    maskc = jnp.zeros((P,), jnp.float32).at[pos2].add(1.0)

    zcol = jnp.zeros((NB, BN, 1), jnp.float32)
    z128 = jnp.zeros((1, 128), jnp.float32)
    cntx2 = jnp.stack([cntx, jnp.zeros_like(cntx)])
    dega = deg1.reshape(NB, BN, 1)
    d2a = deg2.reshape(NB, BN, 1)
    ma = maskc.reshape(NB, BN, 1)

    emb1p = jnp.zeros((128, 32), jnp.float32).at[:101].set(emb1_w)
    x3 = x.reshape(NB, BN, 1)

    gab, dis1 = _tc_prep(x3, dega, zcol, cntx2, emb1p,
                         gn_e1_w.reshape(1, 32), gn_e1_b.reshape(1, 32),
                         gn_e1_ms.reshape(1, 32), W1)

    gflat = gab.reshape(2 * N, 16)
    src1, dst1 = edge1[0], edge1[1]
    acc1 = jnp.zeros((2, N, 16), jnp.float32)
    acc1 = acc1.at[0, :, :].add(
        jnp.zeros((N, 16), jnp.float32).at[dst1].add(gflat[src1]))
    acc1 = acc1.at[1, :, :].add(
        jnp.zeros((N, 16), jnp.float32).at[dst1].add(gflat[N + src1]))
    acc1 = acc1.reshape(2, NB, BN, 16)

    hn, c_arr = _tc_epi(acc1, gab, dis1, _r2d(b1), _r2d(gn1_w),
                        _r2d(gn1_b), _r2d(gn1_ms), ma, zcol)

    hnflat = hn.reshape(2 * N, 16)
    h2a = hnflat[pos1[:, 0]] + hnflat[pos1[:, 1]]
    h2b = hnflat[N + pos1[:, 0]] + hnflat[N + pos1[:, 1]]
    h2ab = jnp.stack([h2a, h2b]).reshape(2, NB, BN, 16)

    g2ab, dis2 = _tc_mid(h2ab, ma, zcol, c_arr, d2a, zcol, emb2_w,
                         gn_e2_w.reshape(1, 16), gn_e2_b.reshape(1, 16),
                         gn_e2_ms.reshape(1, 16), W2)

    g2flat = g2ab.reshape(2 * P, 16)
    src2, dst2 = edge2[0], edge2[1]
    acc2 = jnp.zeros((2, P, 16), jnp.float32)
    acc2 = acc2.at[0, :, :].add(
        jnp.zeros((P, 16), jnp.float32).at[dst2].add(g2flat[src2]))
    acc2 = acc2.at[1, :, :].add(
        jnp.zeros((P, 16), jnp.float32).at[dst2].add(g2flat[P + src2]))
    acc2 = acc2.reshape(2, NB, BN, 16)

    proj, = _tc_epi2(acc2, g2ab, dis2, _r2d(b2), _r2d(gn2_w), _r2d(gn2_b),
                     _r2d(gn2_ms), Wp, bp.reshape(1, 1))

    return proj.reshape(P)[pos2].reshape(Q, 1)


# speedup vs baseline: 12.1061x; 6.6071x over previous
import functools

import jax
import jax.numpy as jnp
from jax import lax
from jax.experimental import pallas as pl
from jax.experimental.pallas import tpu as pltpu
from jax.experimental.pallas import tpu_sc as plsc

N = 100000
E = 1600000
P = 100000
Q = 10000
NB = 100
BN = 1000
EPS = 1e-5

SC_CORES = 2
SC_TILES = 16
EPAD = 1605632
ER = EPAD // 128
ERT = ER // SC_TILES
ECH = 8
NROWS = 100096
ZRT = NROWS // SC_TILES

_SC_MESH = plsc.VectorSubcoreMesh(
    core_axis_name="c", subcore_axis_name="s",
    num_cores=SC_CORES, num_subcores=SC_TILES)


def _edge_body(g_hbm, src_hbm, dst_hbm, zeros_hbm, out_hbm,
               src_v, dst_v, gidx_v, rows_v, acc_sh, sem):
    c = lax.axis_index("c")
    s = lax.axis_index("s")
    zstart = pl.multiple_of(s * ZRT, 8)
    pltpu.sync_copy(zeros_hbm.at[pl.ds(zstart, ZRT)],
                    acc_sh.at[pl.ds(zstart, ZRT)])
    plsc.subcore_barrier()
    coff = c * N
    base = s * ERT

    def chunk(k, carry):
        r0 = pl.multiple_of(base + k * ECH, 8)
        pltpu.sync_copy(src_hbm.at[pl.ds(r0, ECH)], src_v)
        pltpu.sync_copy(dst_hbm.at[pl.ds(r0, ECH)], dst_v)
        for j in range(ECH):
            for q in range(8):
                gidx_v[j, pl.ds(q * 16, 16)] = (
                    src_v[j, pl.ds(q * 16, 16)] + coff)
        descs = [
            pltpu.async_copy(g_hbm.at[gidx_v.at[j]],
                             rows_v.at[pl.ds(j * 128, 128)], sem)
            for j in range(ECH)
        ]
        for d in descs:
            d.wait()
        for j in range(ECH):
            pltpu.sync_copy(rows_v.at[pl.ds(j * 128, 128)],
                            acc_sh.at[dst_v.at[j]], add=True)
        return carry

    lax.fori_loop(0, ERT // ECH, chunk, 0)
    plsc.subcore_barrier()
    wstart = pl.multiple_of(jnp.minimum(s * ZRT, N - ZRT), 8)
    pltpu.sync_copy(acc_sh.at[pl.ds(wstart, ZRT)],
                    out_hbm.at[c, pl.ds(wstart, ZRT)])


_sc_edge = pl.kernel(
    _edge_body,
    out_type=jax.ShapeDtypeStruct((2, N, 16), jnp.float32),
    mesh=_SC_MESH,
    scratch_types=[
        pltpu.VMEM((ECH, 128), jnp.int32),
        pltpu.VMEM((ECH, 128), jnp.int32),
        pltpu.VMEM((ECH, 128), jnp.int32),
        pltpu.VMEM((ECH * 128, 16), jnp.float32),
        pltpu.VMEM_SHARED((NROWS, 16), jnp.float32),
        pltpu.SemaphoreType.DMA,
    ],
    compiler_params=pltpu.CompilerParams(use_tc_tiling_on_sc=False),
)


def _prep_body(x_ref, dega_ref, degb_ref, cntx_ref, emb_ref, gw_ref, gb_ref,
               gms_ref, w1_ref, gab_ref, dis_ref):
    cnt = cntx_ref[0:1, :] + cntx_ref[1:2, :]
    p = cnt / N
    emb = emb_ref[...]
    mean = p @ emb
    ctr = emb - mean * gms_ref[...]
    var = p @ (ctr * ctr)
    tbl = ctr * (gw_ref[...] * lax.rsqrt(var + EPS)) + gb_ref[...]
    t1 = tbl @ w1_ref[...]

    deg = dega_ref[0] + degb_ref[0] + 1.0
    dis = lax.rsqrt(deg)
    dis_ref[0] = dis

    xb = x_ref[0]
    iota = lax.broadcasted_iota(jnp.int32, (BN, 128), 1)
    oh = (xb == iota).astype(jnp.float32)
    g32 = (oh @ t1) * dis
    gab_ref[0, 0] = g32[:, :16]
    gab_ref[1, 0] = g32[:, 16:]


def _tc_prep(x3, dega, degb, cntx, emb1p, gw, gb, gms, w1):
    return pl.pallas_call(
        _prep_body,
        grid=(NB,),
        in_specs=[
            pl.BlockSpec((1, BN, 1), lambda i: (i, 0, 0)),
            pl.BlockSpec((1, BN, 1), lambda i: (i, 0, 0)),
            pl.BlockSpec((1, BN, 1), lambda i: (i, 0, 0)),
            pl.BlockSpec((2, 128), lambda i: (0, 0)),
            pl.BlockSpec((128, 32), lambda i: (0, 0)),
            pl.BlockSpec((1, 32), lambda i: (0, 0)),
            pl.BlockSpec((1, 32), lambda i: (0, 0)),
            pl.BlockSpec((1, 32), lambda i: (0, 0)),
            pl.BlockSpec((32, 32), lambda i: (0, 0)),
        ],
        out_specs=[
            pl.BlockSpec((2, 1, BN, 16), lambda i: (0, i, 0, 0)),
            pl.BlockSpec((1, BN, 1), lambda i: (i, 0, 0)),
        ],
        out_shape=[
            jax.ShapeDtypeStruct((2, NB, BN, 16), jnp.float32),
            jax.ShapeDtypeStruct((NB, BN, 1), jnp.float32),
        ],
    )(x3, dega, degb, cntx, emb1p, gw, gb, gms, w1)


def _epi1_body(acc_ref, g_ref, dis_ref, b_ref, gw_ref, gb_ref, gms_ref,
               ma_ref, mb_ref, hn_ref, c_ref, ssum, ssq, msum):
    ph = pl.program_id(0)
    i = pl.program_id(1)
    nn = jnp.float32(N)

    pre = dis_ref[0] * (acc_ref[:, 0] + g_ref[:, 0]) + b_ref[...][:, None, :]

    @pl.when(jnp.logical_and(ph == 0, i == 0))
    def _init():
        ssum[...] = jnp.zeros_like(ssum)
        ssq[...] = jnp.zeros_like(ssq)
        msum[0, 0] = 0.0

    @pl.when(ph == 0)
    def _accum():
        ssum[...] += jnp.sum(pre, axis=1)
        ssq[...] += jnp.sum(pre * pre, axis=1)
        m = jnp.minimum(ma_ref[0] + mb_ref[0], 1.0)
        msum[0, 0] += jnp.sum(m)

    @pl.when(ph == 1)
    def _apply():
        mean = ssum[...] / nn
        mms = mean * gms_ref[...]
        var = ssq[...] / nn - 2.0 * mms * mean + mms * mms
        scale = gw_ref[...] * lax.rsqrt(var + EPS)
        hn = jnp.maximum((pre - mms[:, None, :]) * scale[:, None, :]
                         + gb_ref[...][:, None, :], 0.0)
        hn_ref[:, 0] = hn

    c_ref[...] = jnp.full((1, 1), msum[0, 0], jnp.float32)


def _tc_epi(acc, g, dis, b2d, gw2d, gb2d, gms2d, ma, mb):
    return pl.pallas_call(
        _epi1_body,
        grid=(2, NB),
        in_specs=[
            pl.BlockSpec((2, 1, BN, 16), lambda p, i: (0, i, 0, 0)),
            pl.BlockSpec((2, 1, BN, 16), lambda p, i: (0, i, 0, 0)),
            pl.BlockSpec((1, BN, 1), lambda p, i: (i, 0, 0)),
            pl.BlockSpec((2, 16), lambda p, i: (0, 0)),
            pl.BlockSpec((2, 16), lambda p, i: (0, 0)),
            pl.BlockSpec((2, 16), lambda p, i: (0, 0)),
            pl.BlockSpec((2, 16), lambda p, i: (0, 0)),
            pl.BlockSpec((1, BN, 1), lambda p, i: (i, 0, 0)),
            pl.BlockSpec((1, BN, 1), lambda p, i: (i, 0, 0)),
        ],
        out_specs=[
            pl.BlockSpec((2, 1, BN, 16), lambda p, i: (0, i, 0, 0)),
            pl.BlockSpec((1, 1), lambda p, i: (0, 0)),
        ],
        out_shape=[
            jax.ShapeDtypeStruct((2, NB, BN, 16), jnp.float32),
            jax.ShapeDtypeStruct((1, 1), jnp.float32),
        ],
        scratch_shapes=[
            pltpu.VMEM((2, 16), jnp.float32),
            pltpu.VMEM((2, 16), jnp.float32),
            pltpu.SMEM((1, 1), jnp.float32),
        ],
    )(acc, g, dis, b2d, gw2d, gb2d, gms2d, ma, mb)


def _mid_body(h2_ref, ma_ref, mb_ref, c_ref, d2a_ref, d2b_ref, emb2_ref,
              zw_ref, zb_ref, zms_ref, w2_ref, g2_ref, dis2_ref):
    pp = jnp.float32(P)
    c = c_ref[...]
    pz1 = c / pp
    pz0 = 1.0 - pz1
    e = emb2_ref[...]
    meanz = pz0 * e[0:1] + pz1 * e[1:2]
    ctr = e - meanz * zms_ref[...]
    varz = pz0 * (ctr[0:1] * ctr[0:1]) + pz1 * (ctr[1:2] * ctr[1:2])
    ztbl = ctr * (zw_ref[...] * lax.rsqrt(varz + EPS)) + zb_ref[...]
    zrow = ztbl @ w2_ref[32:, :]

    h2cat = jnp.concatenate([h2_ref[0, 0], h2_ref[1, 0]], axis=-1)
    m = jnp.minimum(ma_ref[0] + mb_ref[0], 1.0)
    u = h2cat @ w2_ref[:32, :] + zrow[0:1] + m * (zrow[1:2] - zrow[0:1])

    deg = d2a_ref[0] + d2b_ref[0] + 1.0
    dis = lax.rsqrt(deg)
    dis2_ref[0] = dis
    g2 = u * dis
    g2_ref[0, 0] = g2[:, :16]
    g2_ref[1, 0] = g2[:, 16:]


def _tc_mid(h2ab, ma, mb, c_arr, d2a, d2b, emb2, zw, zb, zms, w2):
    return pl.pallas_call(
        _mid_body,
        grid=(NB,),
        in_specs=[
            pl.BlockSpec((2, 1, BN, 16), lambda i: (0, i, 0, 0)),
            pl.BlockSpec((1, BN, 1), lambda i: (i, 0, 0)),
            pl.BlockSpec((1, BN, 1), lambda i: (i, 0, 0)),
            pl.BlockSpec((1, 1), lambda i: (0, 0)),
            pl.BlockSpec((1, BN, 1), lambda i: (i, 0, 0)),
            pl.BlockSpec((1, BN, 1), lambda i: (i, 0, 0)),
            pl.BlockSpec((2, 16), lambda i: (0, 0)),
            pl.BlockSpec((1, 16), lambda i: (0, 0)),
            pl.BlockSpec((1, 16), lambda i: (0, 0)),
            pl.BlockSpec((1, 16), lambda i: (0, 0)),
            pl.BlockSpec((48, 32), lambda i: (0, 0)),
        ],
        out_specs=[
            pl.BlockSpec((2, 1, BN, 16), lambda i: (0, i, 0, 0)),
            pl.BlockSpec((1, BN, 1), lambda i: (i, 0, 0)),
        ],
        out_shape=[
            jax.ShapeDtypeStruct((2, NB, BN, 16), jnp.float32),
            jax.ShapeDtypeStruct((NB, BN, 1), jnp.float32),
        ],
    )(h2ab, ma, mb, c_arr, d2a, d2b, emb2, zw, zb, zms, w2)


def _epi2_body(acc_ref, g_ref, dis_ref, b_ref, gw_ref, gb_ref, gms_ref,
               wp_ref, bp_ref, proj_ref, ssum, ssq):
    ph = pl.program_id(0)
    i = pl.program_id(1)
    nn = jnp.float32(P)

    pre = dis_ref[0] * (acc_ref[:, 0] + g_ref[:, 0]) + b_ref[...][:, None, :]

    @pl.when(jnp.logical_and(ph == 0, i == 0))
    def _init():
        ssum[...] = jnp.zeros_like(ssum)
        ssq[...] = jnp.zeros_like(ssq)

    @pl.when(ph == 0)
    def _accum():
        ssum[...] += jnp.sum(pre, axis=1)
        ssq[...] += jnp.sum(pre * pre, axis=1)

    @pl.when(ph == 1)
    def _apply():
        mean = ssum[...] / nn
        mms = mean * gms_ref[...]
        var = ssq[...] / nn - 2.0 * mms * mean + mms * mms
        scale = gw_ref[...] * lax.rsqrt(var + EPS)
        hf = jnp.maximum((pre - mms[:, None, :]) * scale[:, None, :]
                         + gb_ref[...][:, None, :], 0.0)
        hcat = jnp.concatenate([hf[0], hf[1]], axis=-1)
        proj_ref[0] = hcat @ wp_ref[...] + bp_ref[...]


def _tc_epi2(acc, g, dis, b2d, gw2d, gb2d, gms2d, wp, bp):
    return pl.pallas_call(
        _epi2_body,
        grid=(2, NB),
        in_specs=[
            pl.BlockSpec((2, 1, BN, 16), lambda p, i: (0, i, 0, 0)),
            pl.BlockSpec((2, 1, BN, 16), lambda p, i: (0, i, 0, 0)),
            pl.BlockSpec((1, BN, 1), lambda p, i: (i, 0, 0)),
            pl.BlockSpec((2, 16), lambda p, i: (0, 0)),
            pl.BlockSpec((2, 16), lambda p, i: (0, 0)),
            pl.BlockSpec((2, 16), lambda p, i: (0, 0)),
            pl.BlockSpec((2, 16), lambda p, i: (0, 0)),
            pl.BlockSpec((32, 1), lambda p, i: (0, 0)),
            pl.BlockSpec((1, 1), lambda p, i: (0, 0)),
        ],
        out_specs=[
            pl.BlockSpec((1, BN, 1), lambda p, i: (i, 0, 0)),
        ],
        out_shape=[
            jax.ShapeDtypeStruct((NB, BN, 1), jnp.float32),
        ],
        scratch_shapes=[
            pltpu.VMEM((2, 16), jnp.float32),
            pltpu.VMEM((2, 16), jnp.float32),
        ],
    )(acc, g, dis, b2d, gw2d, gb2d, gms2d, wp, bp)


def _r2d(v):
    return v.reshape(2, 16)


def kernel(x, edge1, edge2, pos1, pos2, emb1_w, emb2_w, gn_e1_w, gn_e1_b,
           gn_e1_ms, gn_e2_w, gn_e2_b, gn_e2_ms, W1, b1, gn1_w, gn1_b,
           gn1_ms, W2, b2, gn2_w, gn2_b, gn2_ms, Wp, bp):
    x = x.astype(jnp.int32)
    edge1 = edge1.astype(jnp.int32)
    edge2 = edge2.astype(jnp.int32)
    pos1 = pos1.astype(jnp.int32)
    pos2 = pos2.astype(jnp.int32)

    cntx = jnp.zeros((128,), jnp.float32).at[x].add(1.0)
    deg1 = jnp.zeros((N,), jnp.float32).at[edge1[1]].add(1.0)
    deg2 = jnp.zeros((P,), jnp.float32).at[edge2[1]].add(1.0)
    maskc = jnp.zeros((P,), jnp.float32).at[pos2].add(1.0)

    zcol = jnp.zeros((NB, BN, 1), jnp.float32)
    z128 = jnp.zeros((1, 128), jnp.float32)
    cntx2 = jnp.stack([cntx, jnp.zeros_like(cntx)])
    dega = deg1.reshape(NB, BN, 1)
    d2a = deg2.reshape(NB, BN, 1)
    ma = maskc.reshape(NB, BN, 1)

    emb1p = jnp.zeros((128, 32), jnp.float32).at[:101].set(emb1_w)
    x3 = x.reshape(NB, BN, 1)

    gab, dis1 = _tc_prep(x3, dega, zcol, cntx2, emb1p,
                         gn_e1_w.reshape(1, 32), gn_e1_b.reshape(1, 32),
                         gn_e1_ms.reshape(1, 32), W1)

    pad_src = (jnp.arange(EPAD - E, dtype=jnp.int32) * 17) % N
    pad_dst = jnp.full((EPAD - E,), N, jnp.int32)
    src1p = jnp.concatenate([edge1[0], pad_src]).reshape(ER, 128)
    dst1p = jnp.concatenate([edge1[1], pad_dst]).reshape(ER, 128)
    src2p = jnp.concatenate([edge2[0], pad_src]).reshape(ER, 128)
    dst2p = jnp.concatenate([edge2[1], pad_dst]).reshape(ER, 128)
    zeros16 = jnp.zeros((NROWS, 16), jnp.float32)

    gflat = gab.reshape(2 * N, 16)
    acc1 = _sc_edge(gflat, src1p, dst1p, zeros16).reshape(2, NB, BN, 16)

    hn, c_arr = _tc_epi(acc1, gab, dis1, _r2d(b1), _r2d(gn1_w),
                        _r2d(gn1_b), _r2d(gn1_ms), ma, zcol)

    hnflat = hn.reshape(2 * N, 16)
    h2a = hnflat[pos1[:, 0]] + hnflat[pos1[:, 1]]
    h2b = hnflat[N + pos1[:, 0]] + hnflat[N + pos1[:, 1]]
    h2ab = jnp.stack([h2a, h2b]).reshape(2, NB, BN, 16)

    g2ab, dis2 = _tc_mid(h2ab, ma, zcol, c_arr, d2a, zcol, emb2_w,
                         gn_e2_w.reshape(1, 16), gn_e2_b.reshape(1, 16),
                         gn_e2_ms.reshape(1, 16), W2)

    g2flat = g2ab.reshape(2 * P, 16)
    acc2 = _sc_edge(g2flat, src2p, dst2p, zeros16).reshape(2, NB, BN, 16)

    proj, = _tc_epi2(acc2, g2ab, dis2, _r2d(b2), _r2d(gn2_w), _r2d(gn2_b),
                     _r2d(gn2_ms), Wp, bp.reshape(1, 1))

    return proj.reshape(P)[pos2].reshape(Q, 1)


# speedup vs baseline: 25.1551x; 2.0779x over previous
import functools

import jax
import jax.numpy as jnp
from jax import lax
from jax.experimental import pallas as pl
from jax.experimental.pallas import tpu as pltpu
from jax.experimental.pallas import tpu_sc as plsc

N = 100000
E = 1600000
P = 100000
Q = 10000
NB = 100
BN = 1000
EPS = 1e-5

SC_CORES = 2
SC_TILES = 16
EPAD = 1605632
ER = EPAD // 128
ERT = ER // SC_TILES
ECH = 8
NROWS = 100096
ZRT = NROWS // SC_TILES

_SC_MESH = plsc.VectorSubcoreMesh(
    core_axis_name="c", subcore_axis_name="s",
    num_cores=SC_CORES, num_subcores=SC_TILES)


def _edge_body(g_hbm, src_hbm, dst_hbm, zeros_hbm, out_hbm,
               src_v, dst_v, gidx_v, rows_v, acc_sh, sem):
    c = lax.axis_index("c")
    s = lax.axis_index("s")
    zstart = pl.multiple_of(s * ZRT, 8)
    pltpu.sync_copy(zeros_hbm.at[pl.ds(zstart, ZRT)],
                    acc_sh.at[pl.ds(zstart, ZRT)])
    plsc.subcore_barrier()
    coff = c * N
    base = s * ERT

    def chunk(k, carry):
        r0 = pl.multiple_of(base + k * ECH, 8)
        pltpu.sync_copy(src_hbm.at[pl.ds(r0, ECH)], src_v)
        pltpu.sync_copy(dst_hbm.at[pl.ds(r0, ECH)], dst_v)
        for j in range(ECH):
            for q in range(8):
                gidx_v[j, pl.ds(q * 16, 16)] = (
                    src_v[j, pl.ds(q * 16, 16)] + coff)
        descs = [
            pltpu.async_copy(g_hbm.at[gidx_v.at[j]],
                             rows_v.at[pl.ds(j * 128, 128)], sem)
            for j in range(ECH)
        ]
        for d in descs:
            d.wait()
        for j in range(ECH):
            pltpu.sync_copy(rows_v.at[pl.ds(j * 128, 128)],
                            acc_sh.at[dst_v.at[j]], add=True)
        return carry

    lax.fori_loop(0, ERT // ECH, chunk, 0)
    plsc.subcore_barrier()
    wstart = pl.multiple_of(jnp.minimum(s * ZRT, N - ZRT), 8)
    pltpu.sync_copy(acc_sh.at[pl.ds(wstart, ZRT)],
                    out_hbm.at[c, pl.ds(wstart, ZRT)])


_sc_edge = pl.kernel(
    _edge_body,
    out_type=jax.ShapeDtypeStruct((2, N, 16), jnp.float32),
    mesh=_SC_MESH,
    scratch_types=[
        pltpu.VMEM((ECH, 128), jnp.int32),
        pltpu.VMEM((ECH, 128), jnp.int32),
        pltpu.VMEM((ECH, 128), jnp.int32),
        pltpu.VMEM((ECH * 128, 16), jnp.float32),
        pltpu.VMEM_SHARED((NROWS, 16), jnp.float32),
        pltpu.SemaphoreType.DMA,
    ],
    compiler_params=pltpu.CompilerParams(use_tc_tiling_on_sc=False),
)


XR = 1024
PR = 256
DRT = ER // 2 // SC_TILES
XRT = XR // 2 // SC_TILES
PRT = PR // 2 // SC_TILES


def _hist_job(idx_hbm, acc_sh, ones_v, idxv, core, s, rows_per_tile):
    base = core * (rows_per_tile * SC_TILES) + s * rows_per_tile

    def chunk(k, carry):
        r0 = pl.multiple_of(base + k * 8, 8)
        pltpu.sync_copy(idx_hbm.at[pl.ds(r0, 8)], idxv)
        for j in range(8):
            pltpu.sync_copy(ones_v, acc_sh.at[idxv.at[j]], add=True)
        return carry

    lax.fori_loop(0, rows_per_tile // 8, chunk, 0)


def _hist_body(dst1_hbm, dst2_hbm, x_hbm, pos2_hbm, zeros_hbm,
               d1_out, d2_out, cnt_out, msk_out,
               idxv, ones_v, acc1_sh, acc2_sh, cntx_sh, mask_sh):
    c = lax.axis_index("c")
    s = lax.axis_index("s")
    for q in range(8):
        ones_v[pl.ds(q * 16, 16)] = jnp.ones((16,), jnp.float32)
    zs = pl.multiple_of(s * ZRT, 8)
    pltpu.sync_copy(zeros_hbm.at[pl.ds(zs, ZRT)], acc1_sh.at[pl.ds(zs, ZRT)])
    pltpu.sync_copy(zeros_hbm.at[pl.ds(zs, ZRT)], acc2_sh.at[pl.ds(zs, ZRT)])
    pltpu.sync_copy(zeros_hbm.at[pl.ds(zs, ZRT)], mask_sh.at[pl.ds(zs, ZRT)])

    @pl.when(s == 0)
    def _zc():
        pltpu.sync_copy(zeros_hbm.at[pl.ds(0, 128)], cntx_sh)

    plsc.subcore_barrier()
    _hist_job(dst1_hbm, acc1_sh, ones_v, idxv, c, s, DRT)
    _hist_job(dst2_hbm, acc2_sh, ones_v, idxv, c, s, DRT)
    _hist_job(x_hbm, cntx_sh, ones_v, idxv, c, s, XRT)
    _hist_job(pos2_hbm, mask_sh, ones_v, idxv, c, s, PRT)
    plsc.subcore_barrier()
    ws = pl.multiple_of(s * ZRT, 8)
    pltpu.sync_copy(acc1_sh.at[pl.ds(ws, ZRT)], d1_out.at[c, pl.ds(ws, ZRT)])
    pltpu.sync_copy(acc2_sh.at[pl.ds(ws, ZRT)], d2_out.at[c, pl.ds(ws, ZRT)])
    pltpu.sync_copy(mask_sh.at[pl.ds(ws, ZRT)], msk_out.at[c, pl.ds(ws, ZRT)])

    @pl.when(s == 0)
    def _wc():
        pltpu.sync_copy(cntx_sh, cnt_out.at[c])


_sc_hist = pl.kernel(
    _hist_body,
    out_type=[
        jax.ShapeDtypeStruct((2, NROWS), jnp.float32),
        jax.ShapeDtypeStruct((2, NROWS), jnp.float32),
        jax.ShapeDtypeStruct((2, 128), jnp.float32),
        jax.ShapeDtypeStruct((2, NROWS), jnp.float32),
    ],
    mesh=_SC_MESH,
    scratch_types=[
        pltpu.VMEM((8, 128), jnp.int32),
        pltpu.VMEM((128,), jnp.float32),
        pltpu.VMEM_SHARED((NROWS,), jnp.float32),
        pltpu.VMEM_SHARED((NROWS,), jnp.float32),
        pltpu.VMEM_SHARED((128,), jnp.float32),
        pltpu.VMEM_SHARED((NROWS,), jnp.float32),
    ],
    compiler_params=pltpu.CompilerParams(use_tc_tiling_on_sc=False),
)


def _prep_body(x_ref, dega_ref, degb_ref, cntx_ref, emb_ref, gw_ref, gb_ref,
               gms_ref, w1_ref, gab_ref, dis_ref):
    cnt = cntx_ref[0:1, :] + cntx_ref[1:2, :]
    p = cnt / N
    emb = emb_ref[...]
    mean = p @ emb
    ctr = emb - mean * gms_ref[...]
    var = p @ (ctr * ctr)
    tbl = ctr * (gw_ref[...] * lax.rsqrt(var + EPS)) + gb_ref[...]
    t1 = tbl @ w1_ref[...]

    deg = dega_ref[0] + degb_ref[0] + 1.0
    dis = lax.rsqrt(deg)
    dis_ref[0] = dis

    xb = x_ref[0]
    iota = lax.broadcasted_iota(jnp.int32, (BN, 128), 1)
    oh = (xb == iota).astype(jnp.float32)
    g32 = (oh @ t1) * dis
    gab_ref[0, 0] = g32[:, :16]
    gab_ref[1, 0] = g32[:, 16:]


def _tc_prep(x3, dega, degb, cntx, emb1p, gw, gb, gms, w1):
    return pl.pallas_call(
        _prep_body,
        grid=(NB,),
        in_specs=[
            pl.BlockSpec((1, BN, 1), lambda i: (i, 0, 0)),
            pl.BlockSpec((1, BN, 1), lambda i: (i, 0, 0)),
            pl.BlockSpec((1, BN, 1), lambda i: (i, 0, 0)),
            pl.BlockSpec((2, 128), lambda i: (0, 0)),
            pl.BlockSpec((128, 32), lambda i: (0, 0)),
            pl.BlockSpec((1, 32), lambda i: (0, 0)),
            pl.BlockSpec((1, 32), lambda i: (0, 0)),
            pl.BlockSpec((1, 32), lambda i: (0, 0)),
            pl.BlockSpec((32, 32), lambda i: (0, 0)),
        ],
        out_specs=[
            pl.BlockSpec((2, 1, BN, 16), lambda i: (0, i, 0, 0)),
            pl.BlockSpec((1, BN, 1), lambda i: (i, 0, 0)),
        ],
        out_shape=[
            jax.ShapeDtypeStruct((2, NB, BN, 16), jnp.float32),
            jax.ShapeDtypeStruct((NB, BN, 1), jnp.float32),
        ],
    )(x3, dega, degb, cntx, emb1p, gw, gb, gms, w1)


def _epi1_body(acc_ref, g_ref, dis_ref, b_ref, gw_ref, gb_ref, gms_ref,
               ma_ref, mb_ref, hn_ref, c_ref, ssum, ssq, msum):
    ph = pl.program_id(0)
    i = pl.program_id(1)
    nn = jnp.float32(N)

    pre = dis_ref[0] * (acc_ref[:, 0] + g_ref[:, 0]) + b_ref[...][:, None, :]

    @pl.when(jnp.logical_and(ph == 0, i == 0))
    def _init():
        ssum[...] = jnp.zeros_like(ssum)
        ssq[...] = jnp.zeros_like(ssq)
        msum[0, 0] = 0.0

    @pl.when(ph == 0)
    def _accum():
        ssum[...] += jnp.sum(pre, axis=1)
        ssq[...] += jnp.sum(pre * pre, axis=1)
        m = jnp.minimum(ma_ref[0] + mb_ref[0], 1.0)
        msum[0, 0] += jnp.sum(m)

    @pl.when(ph == 1)
    def _apply():
        mean = ssum[...] / nn
        mms = mean * gms_ref[...]
        var = ssq[...] / nn - 2.0 * mms * mean + mms * mms
        scale = gw_ref[...] * lax.rsqrt(var + EPS)
        hn = jnp.maximum((pre - mms[:, None, :]) * scale[:, None, :]
                         + gb_ref[...][:, None, :], 0.0)
        hn_ref[:, 0] = hn

    c_ref[...] = jnp.full((1, 1), msum[0, 0], jnp.float32)


def _tc_epi(acc, g, dis, b2d, gw2d, gb2d, gms2d, ma, mb):
    return pl.pallas_call(
        _epi1_body,
        grid=(2, NB),
        in_specs=[
            pl.BlockSpec((2, 1, BN, 16), lambda p, i: (0, i, 0, 0)),
            pl.BlockSpec((2, 1, BN, 16), lambda p, i: (0, i, 0, 0)),
            pl.BlockSpec((1, BN, 1), lambda p, i: (i, 0, 0)),
            pl.BlockSpec((2, 16), lambda p, i: (0, 0)),
            pl.BlockSpec((2, 16), lambda p, i: (0, 0)),
            pl.BlockSpec((2, 16), lambda p, i: (0, 0)),
            pl.BlockSpec((2, 16), lambda p, i: (0, 0)),
            pl.BlockSpec((1, BN, 1), lambda p, i: (i, 0, 0)),
            pl.BlockSpec((1, BN, 1), lambda p, i: (i, 0, 0)),
        ],
        out_specs=[
            pl.BlockSpec((2, 1, BN, 16), lambda p, i: (0, i, 0, 0)),
            pl.BlockSpec((1, 1), lambda p, i: (0, 0)),
        ],
        out_shape=[
            jax.ShapeDtypeStruct((2, NB, BN, 16), jnp.float32),
            jax.ShapeDtypeStruct((1, 1), jnp.float32),
        ],
        scratch_shapes=[
            pltpu.VMEM((2, 16), jnp.float32),
            pltpu.VMEM((2, 16), jnp.float32),
            pltpu.SMEM((1, 1), jnp.float32),
        ],
    )(acc, g, dis, b2d, gw2d, gb2d, gms2d, ma, mb)


def _mid_body(h2_ref, ma_ref, mb_ref, c_ref, d2a_ref, d2b_ref, emb2_ref,
              zw_ref, zb_ref, zms_ref, w2_ref, g2_ref, dis2_ref):
    pp = jnp.float32(P)
    c = c_ref[...]
    pz1 = c / pp
    pz0 = 1.0 - pz1
    e = emb2_ref[...]
    meanz = pz0 * e[0:1] + pz1 * e[1:2]
    ctr = e - meanz * zms_ref[...]
    varz = pz0 * (ctr[0:1] * ctr[0:1]) + pz1 * (ctr[1:2] * ctr[1:2])
    ztbl = ctr * (zw_ref[...] * lax.rsqrt(varz + EPS)) + zb_ref[...]
    zrow = ztbl @ w2_ref[32:, :]

    h2cat = jnp.concatenate([h2_ref[0, 0], h2_ref[1, 0]], axis=-1)
    m = jnp.minimum(ma_ref[0] + mb_ref[0], 1.0)
    u = h2cat @ w2_ref[:32, :] + zrow[0:1] + m * (zrow[1:2] - zrow[0:1])

    deg = d2a_ref[0] + d2b_ref[0] + 1.0
    dis = lax.rsqrt(deg)
    dis2_ref[0] = dis
    g2 = u * dis
    g2_ref[0, 0] = g2[:, :16]
    g2_ref[1, 0] = g2[:, 16:]


def _tc_mid(h2ab, ma, mb, c_arr, d2a, d2b, emb2, zw, zb, zms, w2):
    return pl.pallas_call(
        _mid_body,
        grid=(NB,),
        in_specs=[
            pl.BlockSpec((2, 1, BN, 16), lambda i: (0, i, 0, 0)),
            pl.BlockSpec((1, BN, 1), lambda i: (i, 0, 0)),
            pl.BlockSpec((1, BN, 1), lambda i: (i, 0, 0)),
            pl.BlockSpec((1, 1), lambda i: (0, 0)),
            pl.BlockSpec((1, BN, 1), lambda i: (i, 0, 0)),
            pl.BlockSpec((1, BN, 1), lambda i: (i, 0, 0)),
            pl.BlockSpec((2, 16), lambda i: (0, 0)),
            pl.BlockSpec((1, 16), lambda i: (0, 0)),
            pl.BlockSpec((1, 16), lambda i: (0, 0)),
            pl.BlockSpec((1, 16), lambda i: (0, 0)),
            pl.BlockSpec((48, 32), lambda i: (0, 0)),
        ],
        out_specs=[
            pl.BlockSpec((2, 1, BN, 16), lambda i: (0, i, 0, 0)),
            pl.BlockSpec((1, BN, 1), lambda i: (i, 0, 0)),
        ],
        out_shape=[
            jax.ShapeDtypeStruct((2, NB, BN, 16), jnp.float32),
            jax.ShapeDtypeStruct((NB, BN, 1), jnp.float32),
        ],
    )(h2ab, ma, mb, c_arr, d2a, d2b, emb2, zw, zb, zms, w2)


def _epi2_body(acc_ref, g_ref, dis_ref, b_ref, gw_ref, gb_ref, gms_ref,
               wp_ref, bp_ref, proj_ref, ssum, ssq):
    ph = pl.program_id(0)
    i = pl.program_id(1)
    nn = jnp.float32(P)

    pre = dis_ref[0] * (acc_ref[:, 0] + g_ref[:, 0]) + b_ref[...][:, None, :]

    @pl.when(jnp.logical_and(ph == 0, i == 0))
    def _init():
        ssum[...] = jnp.zeros_like(ssum)
        ssq[...] = jnp.zeros_like(ssq)

    @pl.when(ph == 0)
    def _accum():
        ssum[...] += jnp.sum(pre, axis=1)
        ssq[...] += jnp.sum(pre * pre, axis=1)

    @pl.when(ph == 1)
    def _apply():
        mean = ssum[...] / nn
        mms = mean * gms_ref[...]
        var = ssq[...] / nn - 2.0 * mms * mean + mms * mms
        scale = gw_ref[...] * lax.rsqrt(var + EPS)
        hf = jnp.maximum((pre - mms[:, None, :]) * scale[:, None, :]
                         + gb_ref[...][:, None, :], 0.0)
        hcat = jnp.concatenate([hf[0], hf[1]], axis=-1)
        proj_ref[0] = hcat @ wp_ref[...] + bp_ref[...]


def _tc_epi2(acc, g, dis, b2d, gw2d, gb2d, gms2d, wp, bp):
    return pl.pallas_call(
        _epi2_body,
        grid=(2, NB),
        in_specs=[
            pl.BlockSpec((2, 1, BN, 16), lambda p, i: (0, i, 0, 0)),
            pl.BlockSpec((2, 1, BN, 16), lambda p, i: (0, i, 0, 0)),
            pl.BlockSpec((1, BN, 1), lambda p, i: (i, 0, 0)),
            pl.BlockSpec((2, 16), lambda p, i: (0, 0)),
            pl.BlockSpec((2, 16), lambda p, i: (0, 0)),
            pl.BlockSpec((2, 16), lambda p, i: (0, 0)),
            pl.BlockSpec((2, 16), lambda p, i: (0, 0)),
            pl.BlockSpec((32, 1), lambda p, i: (0, 0)),
            pl.BlockSpec((1, 1), lambda p, i: (0, 0)),
        ],
        out_specs=[
            pl.BlockSpec((1, BN, 1), lambda p, i: (i, 0, 0)),
        ],
        out_shape=[
            jax.ShapeDtypeStruct((NB, BN, 1), jnp.float32),
        ],
        scratch_shapes=[
            pltpu.VMEM((2, 16), jnp.float32),
            pltpu.VMEM((2, 16), jnp.float32),
        ],
    )(acc, g, dis, b2d, gw2d, gb2d, gms2d, wp, bp)


def _r2d(v):
    return v.reshape(2, 16)


def kernel(x, edge1, edge2, pos1, pos2, emb1_w, emb2_w, gn_e1_w, gn_e1_b,
           gn_e1_ms, gn_e2_w, gn_e2_b, gn_e2_ms, W1, b1, gn1_w, gn1_b,
           gn1_ms, W2, b2, gn2_w, gn2_b, gn2_ms, Wp, bp):
    x = x.astype(jnp.int32)
    edge1 = edge1.astype(jnp.int32)
    edge2 = edge2.astype(jnp.int32)
    pos1 = pos1.astype(jnp.int32)
    pos2 = pos2.astype(jnp.int32)

    pad_src = (jnp.arange(EPAD - E, dtype=jnp.int32) * 17) % N
    pad_dst = N + (jnp.arange(EPAD - E, dtype=jnp.int32) % 64)
    src1p = jnp.concatenate([edge1[0], pad_src]).reshape(ER, 128)
    dst1p = jnp.concatenate([edge1[1], pad_dst]).reshape(ER, 128)
    src2p = jnp.concatenate([edge2[0], pad_src]).reshape(ER, 128)
    dst2p = jnp.concatenate([edge2[1], pad_dst]).reshape(ER, 128)
    zeros16 = jnp.zeros((NROWS, 16), jnp.float32)
    zeros1d = jnp.zeros((NROWS,), jnp.float32)
    xp = jnp.concatenate(
        [x, 101 + (jnp.arange(XR * 128 - N, dtype=jnp.int32) % 27)]
    ).reshape(XR, 128)
    pos2p = jnp.concatenate(
        [pos2, N + (jnp.arange(PR * 128 - Q, dtype=jnp.int32) % 64)]
    ).reshape(PR, 128)

    d1o, d2o, cnto, msko = _sc_hist(dst1p, dst2p, xp, pos2p, zeros1d)

    zcol = jnp.zeros((NB, BN, 1), jnp.float32)
    dega = d1o[0, :N].reshape(NB, BN, 1)
    degb = d1o[1, :N].reshape(NB, BN, 1)
    d2a = d2o[0, :P].reshape(NB, BN, 1)
    d2b = d2o[1, :P].reshape(NB, BN, 1)
    ma = msko[0, :P].reshape(NB, BN, 1)
    mb = msko[1, :P].reshape(NB, BN, 1)

    emb1p = jnp.zeros((128, 32), jnp.float32).at[:101].set(emb1_w)
    x3 = x.reshape(NB, BN, 1)

    gab, dis1 = _tc_prep(x3, dega, degb, cnto, emb1p,
                         gn_e1_w.reshape(1, 32), gn_e1_b.reshape(1, 32),
                         gn_e1_ms.reshape(1, 32), W1)

    gflat = gab.reshape(2 * N, 16)
    acc1 = _sc_edge(gflat, src1p, dst1p, zeros16).reshape(2, NB, BN, 16)

    hn, c_arr = _tc_epi(acc1, gab, dis1, _r2d(b1), _r2d(gn1_w),
                        _r2d(gn1_b), _r2d(gn1_ms), ma, mb)

    hnflat = hn.reshape(2 * N, 16)
    h2a = hnflat[pos1[:, 0]] + hnflat[pos1[:, 1]]
    h2b = hnflat[N + pos1[:, 0]] + hnflat[N + pos1[:, 1]]
    h2ab = jnp.stack([h2a, h2b]).reshape(2, NB, BN, 16)

    g2ab, dis2 = _tc_mid(h2ab, ma, mb, c_arr, d2a, d2b, emb2_w,
                         gn_e2_w.reshape(1, 16), gn_e2_b.reshape(1, 16),
                         gn_e2_ms.reshape(1, 16), W2)

    g2flat = g2ab.reshape(2 * P, 16)
    acc2 = _sc_edge(g2flat, src2p, dst2p, zeros16).reshape(2, NB, BN, 16)

    proj, = _tc_epi2(acc2, g2ab, dis2, _r2d(b2), _r2d(gn2_w), _r2d(gn2_b),
                     _r2d(gn2_ms), Wp, bp.reshape(1, 1))

    return proj.reshape(P)[pos2].reshape(Q, 1)


# speedup vs baseline: 25.3777x; 1.0088x over previous
import functools

import jax
import jax.numpy as jnp
from jax import lax
from jax.experimental import pallas as pl
from jax.experimental.pallas import tpu as pltpu
from jax.experimental.pallas import tpu_sc as plsc

N = 100000
E = 1600000
P = 100000
Q = 10000
NB = 100
BN = 1000
EPS = 1e-5

SC_CORES = 2
SC_TILES = 16
EPAD = 1605632
ER = EPAD // 128
ERT = ER // SC_TILES
ECH = 8
NROWS = 100096
ZRT = NROWS // SC_TILES

_SC_MESH = plsc.VectorSubcoreMesh(
    core_axis_name="c", subcore_axis_name="s",
    num_cores=SC_CORES, num_subcores=SC_TILES)


def _edge_body(g_hbm, src_hbm, dst_hbm, zeros_hbm, out_hbm,
               src_v, dst_v, gidx_v, rows_v, acc_sh, sem):
    c = lax.axis_index("c")
    s = lax.axis_index("s")
    zstart = pl.multiple_of(s * ZRT, 8)
    pltpu.sync_copy(zeros_hbm.at[pl.ds(zstart, ZRT)],
                    acc_sh.at[pl.ds(zstart, ZRT)])
    plsc.subcore_barrier()
    coff = c * N
    base = s * ERT

    def chunk(k, carry):
        r0 = pl.multiple_of(base + k * ECH, 8)
        pltpu.sync_copy(src_hbm.at[pl.ds(r0, ECH)], src_v)
        pltpu.sync_copy(dst_hbm.at[pl.ds(r0, ECH)], dst_v)
        for j in range(ECH):
            for q in range(8):
                gidx_v[j, pl.ds(q * 16, 16)] = (
                    src_v[j, pl.ds(q * 16, 16)] + coff)
        descs = [
            pltpu.async_copy(g_hbm.at[gidx_v.at[j]],
                             rows_v.at[pl.ds(j * 128, 128)], sem)
            for j in range(ECH)
        ]
        for d in descs:
            d.wait()
        for j in range(ECH):
            pltpu.sync_copy(rows_v.at[pl.ds(j * 128, 128)],
                            acc_sh.at[dst_v.at[j]], add=True)
        return carry

    lax.fori_loop(0, ERT // ECH, chunk, 0)
    plsc.subcore_barrier()
    wstart = pl.multiple_of(jnp.minimum(s * ZRT, N - ZRT), 8)
    pltpu.sync_copy(acc_sh.at[pl.ds(wstart, ZRT)],
                    out_hbm.at[c, pl.ds(wstart, ZRT)])


_sc_edge = pl.kernel(
    _edge_body,
    out_type=jax.ShapeDtypeStruct((2, N, 16), jnp.float32),
    mesh=_SC_MESH,
    scratch_types=[
        pltpu.VMEM((ECH, 128), jnp.int32),
        pltpu.VMEM((ECH, 128), jnp.int32),
        pltpu.VMEM((ECH, 128), jnp.int32),
        pltpu.VMEM((ECH * 128, 16), jnp.float32),
        pltpu.VMEM_SHARED((NROWS, 16), jnp.float32),
        pltpu.SemaphoreType.DMA,
    ],
    compiler_params=pltpu.CompilerParams(use_tc_tiling_on_sc=False),
)


XR = 1024
PR = 256
DRT = ER // 2 // SC_TILES
XRT = XR // 2 // SC_TILES
PRT = PR // 2 // SC_TILES


def _hist_job(idx_hbm, acc_sh, ones_v, idxv, core, s, rows_per_tile):
    base = core * (rows_per_tile * SC_TILES) + s * rows_per_tile

    def chunk(k, carry):
        r0 = pl.multiple_of(base + k * 8, 8)
        pltpu.sync_copy(idx_hbm.at[pl.ds(r0, 8)], idxv)
        for j in range(8):
            pltpu.sync_copy(ones_v, acc_sh.at[idxv.at[j]], add=True)
        return carry

    lax.fori_loop(0, rows_per_tile // 8, chunk, 0)


def _hist_body(dst1_hbm, dst2_hbm, x_hbm, pos2_hbm, zeros_hbm,
               d1_out, d2_out, cnt_out, msk_out,
               idxv, ones_v, acc1_sh, acc2_sh, cntx_sh, mask_sh):
    c = lax.axis_index("c")
    s = lax.axis_index("s")
    for q in range(8):
        ones_v[pl.ds(q * 16, 16)] = jnp.ones((16,), jnp.float32)
    zs = pl.multiple_of(s * ZRT, 8)
    pltpu.sync_copy(zeros_hbm.at[pl.ds(zs, ZRT)], acc1_sh.at[pl.ds(zs, ZRT)])
    pltpu.sync_copy(zeros_hbm.at[pl.ds(zs, ZRT)], acc2_sh.at[pl.ds(zs, ZRT)])
    pltpu.sync_copy(zeros_hbm.at[pl.ds(zs, ZRT)], mask_sh.at[pl.ds(zs, ZRT)])

    @pl.when(s == 0)
    def _zc():
        pltpu.sync_copy(zeros_hbm.at[pl.ds(0, 128)], cntx_sh)

    plsc.subcore_barrier()
    _hist_job(dst1_hbm, acc1_sh, ones_v, idxv, c, s, DRT)
    _hist_job(dst2_hbm, acc2_sh, ones_v, idxv, c, s, DRT)
    _hist_job(x_hbm, cntx_sh, ones_v, idxv, c, s, XRT)
    _hist_job(pos2_hbm, mask_sh, ones_v, idxv, c, s, PRT)
    plsc.subcore_barrier()
    ws = pl.multiple_of(s * ZRT, 8)
    pltpu.sync_copy(acc1_sh.at[pl.ds(ws, ZRT)], d1_out.at[c, pl.ds(ws, ZRT)])
    pltpu.sync_copy(acc2_sh.at[pl.ds(ws, ZRT)], d2_out.at[c, pl.ds(ws, ZRT)])
    pltpu.sync_copy(mask_sh.at[pl.ds(ws, ZRT)], msk_out.at[c, pl.ds(ws, ZRT)])

    @pl.when(s == 0)
    def _wc():
        pltpu.sync_copy(cntx_sh, cnt_out.at[c])


_sc_hist = pl.kernel(
    _hist_body,
    out_type=[
        jax.ShapeDtypeStruct((2, NROWS), jnp.float32),
        jax.ShapeDtypeStruct((2, NROWS), jnp.float32),
        jax.ShapeDtypeStruct((2, 128), jnp.float32),
        jax.ShapeDtypeStruct((2, NROWS), jnp.float32),
    ],
    mesh=_SC_MESH,
    scratch_types=[
        pltpu.VMEM((8, 128), jnp.int32),
        pltpu.VMEM((128,), jnp.float32),
        pltpu.VMEM_SHARED((NROWS,), jnp.float32),
        pltpu.VMEM_SHARED((NROWS,), jnp.float32),
        pltpu.VMEM_SHARED((128,), jnp.float32),
        pltpu.VMEM_SHARED((NROWS,), jnp.float32),
    ],
    compiler_params=pltpu.CompilerParams(use_tc_tiling_on_sc=False),
)


PPAD = 114688
PPR = PPAD // 128
PPT = PPR // SC_TILES


def _pool_body(hn_hbm, pa_hbm, pb_hbm, out_hbm,
               idxa, idxb, gidx, bufa, bufb, sem):
    c = lax.axis_index("c")
    s = lax.axis_index("s")
    coff = c * N
    base = s * PPT

    def chunk(k, carry):
        r0 = pl.multiple_of(base + k * 8, 8)
        pltpu.sync_copy(pa_hbm.at[pl.ds(r0, 8)], idxa)
        pltpu.sync_copy(pb_hbm.at[pl.ds(r0, 8)], idxb)
        for j in range(8):
            for q in range(8):
                gidx[j, pl.ds(q * 16, 16)] = idxa[j, pl.ds(q * 16, 16)] + coff
        da = [pltpu.async_copy(hn_hbm.at[gidx.at[j]],
                               bufa.at[pl.ds(j * 128, 128)], sem)
              for j in range(8)]
        for d in da:
            d.wait()
        for j in range(8):
            for q in range(8):
                gidx[j, pl.ds(q * 16, 16)] = idxb[j, pl.ds(q * 16, 16)] + coff
        db = [pltpu.async_copy(hn_hbm.at[gidx.at[j]],
                               bufb.at[pl.ds(j * 128, 128)], sem)
              for j in range(8)]
        for d in db:
            d.wait()

        def addrow(r, carry2):
            for u in range(8):
                row = r * 8 + u
                bufa[row, :] = bufa[row, :] + bufb[row, :]
            return carry2

        lax.fori_loop(0, 128, addrow, 0)
        pltpu.sync_copy(bufa, out_hbm.at[c, pl.ds(pl.multiple_of(r0 * 128, 8), 1024)])
        return carry

    lax.fori_loop(0, PPT // 8, chunk, 0)


_sc_pool = pl.kernel(
    _pool_body,
    out_type=jax.ShapeDtypeStruct((2, PPAD, 16), jnp.float32),
    mesh=_SC_MESH,
    scratch_types=[
        pltpu.VMEM((8, 128), jnp.int32),
        pltpu.VMEM((8, 128), jnp.int32),
        pltpu.VMEM((8, 128), jnp.int32),
        pltpu.VMEM((1024, 16), jnp.float32),
        pltpu.VMEM((1024, 16), jnp.float32),
        pltpu.SemaphoreType.DMA,
    ],
    compiler_params=pltpu.CompilerParams(use_tc_tiling_on_sc=False),
)


QPAD = 32768
QRT = 256 // 2 // SC_TILES


def _gatherq_body(proj_hbm, pos2_hbm, out_hbm, idxv, vals, sem):
    c = lax.axis_index("c")
    s = lax.axis_index("s")
    base = pl.multiple_of(c * (QRT * SC_TILES) + s * QRT, 8)
    pltpu.sync_copy(pos2_hbm.at[pl.ds(base, QRT)], idxv)
    ds_ = [pltpu.async_copy(proj_hbm.at[idxv.at[j]], vals.at[j], sem)
           for j in range(QRT)]
    for d in ds_:
        d.wait()
    pltpu.sync_copy(vals, out_hbm.at[pl.ds(base, QRT)])


_sc_gatherq = pl.kernel(
    _gatherq_body,
    out_type=jax.ShapeDtypeStruct((256, 128), jnp.float32),
    mesh=_SC_MESH,
    scratch_types=[
        pltpu.VMEM((QRT, 128), jnp.int32),
        pltpu.VMEM((QRT, 128), jnp.float32),
        pltpu.SemaphoreType.DMA,
    ],
    compiler_params=pltpu.CompilerParams(use_tc_tiling_on_sc=False),
)


def _prep_body(x_ref, dega_ref, degb_ref, cntx_ref, emb_ref, gw_ref, gb_ref,
               gms_ref, w1_ref, gab_ref, dis_ref):
    cnt = cntx_ref[0:1, :] + cntx_ref[1:2, :]
    p = cnt / N
    emb = emb_ref[...]
    mean = p @ emb
    ctr = emb - mean * gms_ref[...]
    var = p @ (ctr * ctr)
    tbl = ctr * (gw_ref[...] * lax.rsqrt(var + EPS)) + gb_ref[...]
    t1 = tbl @ w1_ref[...]

    deg = dega_ref[0] + degb_ref[0] + 1.0
    dis = lax.rsqrt(deg)
    dis_ref[0] = dis

    xb = x_ref[0]
    iota = lax.broadcasted_iota(jnp.int32, (BN, 128), 1)
    oh = (xb == iota).astype(jnp.float32)
    g32 = (oh @ t1) * dis
    gab_ref[0, 0] = g32[:, :16]
    gab_ref[1, 0] = g32[:, 16:]


def _tc_prep(x3, dega, degb, cntx, emb1p, gw, gb, gms, w1):
    return pl.pallas_call(
        _prep_body,
        grid=(NB,),
        in_specs=[
            pl.BlockSpec((1, BN, 1), lambda i: (i, 0, 0)),
            pl.BlockSpec((1, BN, 1), lambda i: (i, 0, 0)),
            pl.BlockSpec((1, BN, 1), lambda i: (i, 0, 0)),
            pl.BlockSpec((2, 128), lambda i: (0, 0)),
            pl.BlockSpec((128, 32), lambda i: (0, 0)),
            pl.BlockSpec((1, 32), lambda i: (0, 0)),
            pl.BlockSpec((1, 32), lambda i: (0, 0)),
            pl.BlockSpec((1, 32), lambda i: (0, 0)),
            pl.BlockSpec((32, 32), lambda i: (0, 0)),
        ],
        out_specs=[
            pl.BlockSpec((2, 1, BN, 16), lambda i: (0, i, 0, 0)),
            pl.BlockSpec((1, BN, 1), lambda i: (i, 0, 0)),
        ],
        out_shape=[
            jax.ShapeDtypeStruct((2, NB, BN, 16), jnp.float32),
            jax.ShapeDtypeStruct((NB, BN, 1), jnp.float32),
        ],
    )(x3, dega, degb, cntx, emb1p, gw, gb, gms, w1)


def _epi1_body(acc_ref, g_ref, dis_ref, b_ref, gw_ref, gb_ref, gms_ref,
               ma_ref, mb_ref, hn_ref, c_ref, ssum, ssq, msum):
    ph = pl.program_id(0)
    i = pl.program_id(1)
    nn = jnp.float32(N)

    pre = dis_ref[0] * (acc_ref[:, 0] + g_ref[:, 0]) + b_ref[...][:, None, :]

    @pl.when(jnp.logical_and(ph == 0, i == 0))
    def _init():
        ssum[...] = jnp.zeros_like(ssum)
        ssq[...] = jnp.zeros_like(ssq)
        msum[0, 0] = 0.0

    @pl.when(ph == 0)
    def _accum():
        ssum[...] += jnp.sum(pre, axis=1)
        ssq[...] += jnp.sum(pre * pre, axis=1)
        m = jnp.minimum(ma_ref[0] + mb_ref[0], 1.0)
        msum[0, 0] += jnp.sum(m)

    @pl.when(ph == 1)
    def _apply():
        mean = ssum[...] / nn
        mms = mean * gms_ref[...]
        var = ssq[...] / nn - 2.0 * mms * mean + mms * mms
        scale = gw_ref[...] * lax.rsqrt(var + EPS)
        hn = jnp.maximum((pre - mms[:, None, :]) * scale[:, None, :]
                         + gb_ref[...][:, None, :], 0.0)
        hn_ref[:, 0] = hn

    c_ref[...] = jnp.full((1, 1), msum[0, 0], jnp.float32)


def _tc_epi(acc, g, dis, b2d, gw2d, gb2d, gms2d, ma, mb):
    return pl.pallas_call(
        _epi1_body,
        grid=(2, NB),
        in_specs=[
            pl.BlockSpec((2, 1, BN, 16), lambda p, i: (0, i, 0, 0)),
            pl.BlockSpec((2, 1, BN, 16), lambda p, i: (0, i, 0, 0)),
            pl.BlockSpec((1, BN, 1), lambda p, i: (i, 0, 0)),
            pl.BlockSpec((2, 16), lambda p, i: (0, 0)),
            pl.BlockSpec((2, 16), lambda p, i: (0, 0)),
            pl.BlockSpec((2, 16), lambda p, i: (0, 0)),
            pl.BlockSpec((2, 16), lambda p, i: (0, 0)),
            pl.BlockSpec((1, BN, 1), lambda p, i: (i, 0, 0)),
            pl.BlockSpec((1, BN, 1), lambda p, i: (i, 0, 0)),
        ],
        out_specs=[
            pl.BlockSpec((2, 1, BN, 16), lambda p, i: (0, i, 0, 0)),
            pl.BlockSpec((1, 1), lambda p, i: (0, 0)),
        ],
        out_shape=[
            jax.ShapeDtypeStruct((2, NB, BN, 16), jnp.float32),
            jax.ShapeDtypeStruct((1, 1), jnp.float32),
        ],
        scratch_shapes=[
            pltpu.VMEM((2, 16), jnp.float32),
            pltpu.VMEM((2, 16), jnp.float32),
            pltpu.SMEM((1, 1), jnp.float32),
        ],
    )(acc, g, dis, b2d, gw2d, gb2d, gms2d, ma, mb)


def _mid_body(h2_ref, ma_ref, mb_ref, c_ref, d2a_ref, d2b_ref, emb2_ref,
              zw_ref, zb_ref, zms_ref, w2_ref, g2_ref, dis2_ref):
    pp = jnp.float32(P)
    c = c_ref[...]
    pz1 = c / pp
    pz0 = 1.0 - pz1
    e = emb2_ref[...]
    meanz = pz0 * e[0:1] + pz1 * e[1:2]
    ctr = e - meanz * zms_ref[...]
    varz = pz0 * (ctr[0:1] * ctr[0:1]) + pz1 * (ctr[1:2] * ctr[1:2])
    ztbl = ctr * (zw_ref[...] * lax.rsqrt(varz + EPS)) + zb_ref[...]
    zrow = ztbl @ w2_ref[32:, :]

    h2cat = jnp.concatenate([h2_ref[0, 0], h2_ref[1, 0]], axis=-1)
    m = jnp.minimum(ma_ref[0] + mb_ref[0], 1.0)
    u = h2cat @ w2_ref[:32, :] + zrow[0:1] + m * (zrow[1:2] - zrow[0:1])

    deg = d2a_ref[0] + d2b_ref[0] + 1.0
    dis = lax.rsqrt(deg)
    dis2_ref[0] = dis
    g2 = u * dis
    g2_ref[0, 0] = g2[:, :16]
    g2_ref[1, 0] = g2[:, 16:]


def _tc_mid(h2ab, ma, mb, c_arr, d2a, d2b, emb2, zw, zb, zms, w2):
    return pl.pallas_call(
        _mid_body,
        grid=(NB,),
        in_specs=[
            pl.BlockSpec((2, 1, BN, 16), lambda i: (0, i, 0, 0)),
            pl.BlockSpec((1, BN, 1), lambda i: (i, 0, 0)),
            pl.BlockSpec((1, BN, 1), lambda i: (i, 0, 0)),
            pl.BlockSpec((1, 1), lambda i: (0, 0)),
            pl.BlockSpec((1, BN, 1), lambda i: (i, 0, 0)),
            pl.BlockSpec((1, BN, 1), lambda i: (i, 0, 0)),
            pl.BlockSpec((2, 16), lambda i: (0, 0)),
            pl.BlockSpec((1, 16), lambda i: (0, 0)),
            pl.BlockSpec((1, 16), lambda i: (0, 0)),
            pl.BlockSpec((1, 16), lambda i: (0, 0)),
            pl.BlockSpec((48, 32), lambda i: (0, 0)),
        ],
        out_specs=[
            pl.BlockSpec((2, 1, BN, 16), lambda i: (0, i, 0, 0)),
            pl.BlockSpec((1, BN, 1), lambda i: (i, 0, 0)),
        ],
        out_shape=[
            jax.ShapeDtypeStruct((2, NB, BN, 16), jnp.float32),
            jax.ShapeDtypeStruct((NB, BN, 1), jnp.float32),
        ],
    )(h2ab, ma, mb, c_arr, d2a, d2b, emb2, zw, zb, zms, w2)


def _epi2_body(acc_ref, g_ref, dis_ref, b_ref, gw_ref, gb_ref, gms_ref,
               wp_ref, bp_ref, proj_ref, ssum, ssq):
    ph = pl.program_id(0)
    i = pl.program_id(1)
    nn = jnp.float32(P)

    pre = dis_ref[0] * (acc_ref[:, 0] + g_ref[:, 0]) + b_ref[...][:, None, :]

    @pl.when(jnp.logical_and(ph == 0, i == 0))
    def _init():
        ssum[...] = jnp.zeros_like(ssum)
        ssq[...] = jnp.zeros_like(ssq)

    @pl.when(ph == 0)
    def _accum():
        ssum[...] += jnp.sum(pre, axis=1)
        ssq[...] += jnp.sum(pre * pre, axis=1)

    @pl.when(ph == 1)
    def _apply():
        mean = ssum[...] / nn
        mms = mean * gms_ref[...]
        var = ssq[...] / nn - 2.0 * mms * mean + mms * mms
        scale = gw_ref[...] * lax.rsqrt(var + EPS)
        hf = jnp.maximum((pre - mms[:, None, :]) * scale[:, None, :]
                         + gb_ref[...][:, None, :], 0.0)
        hcat = jnp.concatenate([hf[0], hf[1]], axis=-1)
        proj_ref[0] = hcat @ wp_ref[...] + bp_ref[...]


def _tc_epi2(acc, g, dis, b2d, gw2d, gb2d, gms2d, wp, bp):
    return pl.pallas_call(
        _epi2_body,
        grid=(2, NB),
        in_specs=[
            pl.BlockSpec((2, 1, BN, 16), lambda p, i: (0, i, 0, 0)),
            pl.BlockSpec((2, 1, BN, 16), lambda p, i: (0, i, 0, 0)),
            pl.BlockSpec((1, BN, 1), lambda p, i: (i, 0, 0)),
            pl.BlockSpec((2, 16), lambda p, i: (0, 0)),
            pl.BlockSpec((2, 16), lambda p, i: (0, 0)),
            pl.BlockSpec((2, 16), lambda p, i: (0, 0)),
            pl.BlockSpec((2, 16), lambda p, i: (0, 0)),
            pl.BlockSpec((32, 1), lambda p, i: (0, 0)),
            pl.BlockSpec((1, 1), lambda p, i: (0, 0)),
        ],
        out_specs=[
            pl.BlockSpec((1, BN, 1), lambda p, i: (i, 0, 0)),
        ],
        out_shape=[
            jax.ShapeDtypeStruct((NB, BN, 1), jnp.float32),
        ],
        scratch_shapes=[
            pltpu.VMEM((2, 16), jnp.float32),
            pltpu.VMEM((2, 16), jnp.float32),
        ],
    )(acc, g, dis, b2d, gw2d, gb2d, gms2d, wp, bp)


def _r2d(v):
    return v.reshape(2, 16)


def kernel(x, edge1, edge2, pos1, pos2, emb1_w, emb2_w, gn_e1_w, gn_e1_b,
           gn_e1_ms, gn_e2_w, gn_e2_b, gn_e2_ms, W1, b1, gn1_w, gn1_b,
           gn1_ms, W2, b2, gn2_w, gn2_b, gn2_ms, Wp, bp):
    x = x.astype(jnp.int32)
    edge1 = edge1.astype(jnp.int32)
    edge2 = edge2.astype(jnp.int32)
    pos1 = pos1.astype(jnp.int32)
    pos2 = pos2.astype(jnp.int32)

    pad_src = (jnp.arange(EPAD - E, dtype=jnp.int32) * 17) % N
    pad_dst = N + (jnp.arange(EPAD - E, dtype=jnp.int32) % 64)
    src1p = jnp.concatenate([edge1[0], pad_src]).reshape(ER, 128)
    dst1p = jnp.concatenate([edge1[1], pad_dst]).reshape(ER, 128)
    src2p = jnp.concatenate([edge2[0], pad_src]).reshape(ER, 128)
    dst2p = jnp.concatenate([edge2[1], pad_dst]).reshape(ER, 128)
    zeros16 = jnp.zeros((NROWS, 16), jnp.float32)
    zeros1d = jnp.zeros((NROWS,), jnp.float32)
    xp = jnp.concatenate(
        [x, 101 + (jnp.arange(XR * 128 - N, dtype=jnp.int32) % 27)]
    ).reshape(XR, 128)
    pos2p = jnp.concatenate(
        [pos2, N + (jnp.arange(PR * 128 - Q, dtype=jnp.int32) % 64)]
    ).reshape(PR, 128)

    d1o, d2o, cnto, msko = _sc_hist(dst1p, dst2p, xp, pos2p, zeros1d)

    zcol = jnp.zeros((NB, BN, 1), jnp.float32)
    dega = d1o[0, :N].reshape(NB, BN, 1)
    degb = d1o[1, :N].reshape(NB, BN, 1)
    d2a = d2o[0, :P].reshape(NB, BN, 1)
    d2b = d2o[1, :P].reshape(NB, BN, 1)
    ma = msko[0, :P].reshape(NB, BN, 1)
    mb = msko[1, :P].reshape(NB, BN, 1)

    emb1p = jnp.zeros((128, 32), jnp.float32).at[:101].set(emb1_w)
    x3 = x.reshape(NB, BN, 1)

    gab, dis1 = _tc_prep(x3, dega, degb, cnto, emb1p,
                         gn_e1_w.reshape(1, 32), gn_e1_b.reshape(1, 32),
                         gn_e1_ms.reshape(1, 32), W1)

    gflat = gab.reshape(2 * N, 16)
    acc1 = _sc_edge(gflat, src1p, dst1p, zeros16).reshape(2, NB, BN, 16)

    hn, c_arr = _tc_epi(acc1, gab, dis1, _r2d(b1), _r2d(gn1_w),
                        _r2d(gn1_b), _r2d(gn1_ms), ma, mb)

    hnflat = hn.reshape(2 * N, 16)
    pad_pool = (jnp.arange(PPAD - P, dtype=jnp.int32) * 31) % N
    pa = jnp.concatenate([pos1[:, 0], pad_pool]).reshape(PPR, 128)
    pb = jnp.concatenate([pos1[:, 1], pad_pool]).reshape(PPR, 128)
    h2ab = _sc_pool(hnflat, pa, pb)[:, :P].reshape(2, NB, BN, 16)

    g2ab, dis2 = _tc_mid(h2ab, ma, mb, c_arr, d2a, d2b, emb2_w,
                         gn_e2_w.reshape(1, 16), gn_e2_b.reshape(1, 16),
                         gn_e2_ms.reshape(1, 16), W2)

    g2flat = g2ab.reshape(2 * P, 16)
    acc2 = _sc_edge(g2flat, src2p, dst2p, zeros16).reshape(2, NB, BN, 16)

    proj, = _tc_epi2(acc2, g2ab, dis2, _r2d(b2), _r2d(gn2_w), _r2d(gn2_b),
                     _r2d(gn2_ms), Wp, bp.reshape(1, 1))

    pos2g = jnp.concatenate(
        [pos2, (jnp.arange(QPAD - Q, dtype=jnp.int32) * 131) % P]
    ).reshape(256, 128)
    outq = _sc_gatherq(proj.reshape(P), pos2g)
    return outq.reshape(QPAD)[:Q].reshape(Q, 1)
